# Initial kernel scaffold; baseline (speedup 1.0000x reference)
import jax
import jax.numpy as jnp
from jax.experimental import pallas as pl


def kernel(h, x, edge_idx, edge_attr, W_e1, b_e1, W_e2, b_e2, W_att, b_att, W_n1, b_n1, W_n2, b_n2, W_c1, b_c1, W_c2, gamma, beta):
    raise NotImplementedError("write your pallas kernel here")


# speedup vs baseline: 2.5533x; 2.5533x over previous
import functools

import jax
import jax.numpy as jnp
from jax import lax
from jax.experimental import pallas as pl
from jax.experimental.pallas import tpu as pltpu
from jax.experimental.pallas import tpu_sc as plsc

_EPS = 1e-08
_XW = 16
_EB = 80
_NW = 32
_NC = 2


def _silu(v):
    return v * jax.nn.sigmoid(v)


def _precompute_ab(h, w1a, w1b):
    n, d = h.shape
    bn = 1000
    def body(h_ref, wa_ref, wb_ref, a_ref, b_ref):
        hb = h_ref[...]
        a_ref[...] = jnp.dot(hb, wa_ref[...], preferred_element_type=jnp.float32)
        b_ref[...] = jnp.dot(hb, wb_ref[...], preferred_element_type=jnp.float32)
    return pl.pallas_call(
        body,
        grid=(n // bn,),
        in_specs=[
            pl.BlockSpec((bn, d), lambda i: (i, 0)),
            pl.BlockSpec(w1a.shape, lambda i: (0, 0)),
            pl.BlockSpec(w1b.shape, lambda i: (0, 0)),
        ],
        out_specs=[pl.BlockSpec((bn, w1a.shape[1]), lambda i: (i, 0))] * 2,
        out_shape=[jax.ShapeDtypeStruct((n, w1a.shape[1]), jnp.float32)] * 2,
    )(h, w1a, w1b)


def _sc_gather(row, col, a_tab, b_tab, x_tab):
    e = row.shape[0]
    n, hdim = a_tab.shape
    xw = x_tab.shape[1]
    ept = e // _NW
    nch = ept // _EB
    mesh = plsc.VectorSubcoreMesh(core_axis_name="c", subcore_axis_name="s")

    @functools.partial(
        pl.kernel,
        mesh=mesh,
        out_type=(
            jax.ShapeDtypeStruct((e, hdim), jnp.float32),
            jax.ShapeDtypeStruct((e, hdim), jnp.float32),
            jax.ShapeDtypeStruct((e, _XW), jnp.float32),
        ),
        scratch_types=[
            pltpu.VMEM((_EB,), jnp.int32),
            pltpu.VMEM((_EB,), jnp.int32),
            pltpu.VMEM((_EB, hdim), jnp.float32),
            pltpu.VMEM((_EB, hdim), jnp.float32),
            pltpu.VMEM((_EB, xw), jnp.float32),
            pltpu.VMEM((_EB, xw), jnp.float32),
            pltpu.VMEM((_EB, _XW), jnp.float32),
            pltpu.SemaphoreType.DMA,
        ],
    )
    def k(row_h, col_h, a_h, b_h, x_h, ar_o, bc_o, d_o,
          ri, ci, ab, bb, xrb, xcb, df, sem):
        wid = lax.axis_index("s") * _NC + lax.axis_index("c")
        base0 = wid * ept

        def body(j, carry):
            base = base0 + j * _EB
            pltpu.sync_copy(row_h.at[pl.ds(base, _EB)], ri)
            pltpu.sync_copy(col_h.at[pl.ds(base, _EB)], ci)
            c1 = pltpu.async_copy(a_h.at[ri], ab, sem)
            c2 = pltpu.async_copy(b_h.at[ci], bb, sem)
            c3 = pltpu.async_copy(x_h.at[ri], xrb, sem)
            c4 = pltpu.async_copy(x_h.at[ci], xcb, sem)
            c1.wait(); c2.wait(); c3.wait(); c4.wait()

            def drow(i, c):
                df[i] = xrb[i, pl.ds(0, _XW)] - xcb[i, pl.ds(0, _XW)]
                return c
            lax.fori_loop(0, _EB, drow, 0)

            pltpu.sync_copy(ab, ar_o.at[pl.ds(base, _EB)])
            pltpu.sync_copy(bb, bc_o.at[pl.ds(base, _EB)])
            pltpu.sync_copy(df, d_o.at[pl.ds(base, _EB)])
            return carry

        lax.fori_loop(0, nch, body, 0)

    return k(row, col, a_tab, b_tab, x_tab)


def _edge_mlp(arow, bcol, dvec, eattr, w_ea, w_d, b_e1, w_e2, b_e2,
              w_att_r, b_att, w_c1, b_c1, w_c2_r):
    e, hdim = arow.shape
    ed = eattr.shape[1]
    be = 512

    def body(ar_ref, bc_ref, d_ref, ea_ref, wea_ref, wd_ref, be1_ref,
             we2_ref, be2_ref, watt_ref, batt_ref, wc1_ref, bc1_ref, wc2_ref,
             msg_ref, crd_ref):
        d = d_ref[...]
        dist_sq = jnp.sum(d * d, axis=1, keepdims=True)
        pre1 = (ar_ref[...] + bc_ref[...] + dist_sq * wd_ref[...]
                + jnp.dot(ea_ref[...], wea_ref[...],
                          preferred_element_type=jnp.float32) + be1_ref[...])
        t = _silu(pre1)
        msg0 = _silu(jnp.dot(t, we2_ref[...],
                             preferred_element_type=jnp.float32) + be2_ref[...])
        att = jax.nn.sigmoid(
            jnp.sum(msg0 * watt_ref[...], axis=1, keepdims=True) + batt_ref[...])
        msg = msg0 * att
        u = _silu(jnp.dot(msg, wc1_ref[...],
                          preferred_element_type=jnp.float32) + bc1_ref[...])
        cw = jnp.tanh(jnp.sum(u * wc2_ref[...], axis=1, keepdims=True))
        unit = d * jax.lax.rsqrt(dist_sq + _EPS)
        msg_ref[...] = msg
        crd_ref[...] = cw * unit

    full = lambda s: pl.BlockSpec(s, lambda i: (0, 0))
    return pl.pallas_call(
        body,
        grid=(e // be,),
        in_specs=[
            pl.BlockSpec((be, hdim), lambda i: (i, 0)),
            pl.BlockSpec((be, hdim), lambda i: (i, 0)),
            pl.BlockSpec((be, _XW), lambda i: (i, 0)),
            pl.BlockSpec((be, ed), lambda i: (i, 0)),
            full((ed, hdim)), full((1, hdim)), full((1, hdim)),
            full((hdim, hdim)), full((1, hdim)),
            full((1, hdim)), full((1, 1)),
            full((hdim, hdim)), full((1, hdim)), full((1, hdim)),
        ],
        out_specs=[
            pl.BlockSpec((be, hdim), lambda i: (i, 0)),
            pl.BlockSpec((be, _XW), lambda i: (i, 0)),
        ],
        out_shape=[
            jax.ShapeDtypeStruct((e, hdim), jnp.float32),
            jax.ShapeDtypeStruct((e, _XW), jnp.float32),
        ],
    )(arow, bcol, dvec, eattr, w_ea, w_d, b_e1, w_e2, b_e2,
      w_att_r, b_att, w_c1, b_c1, w_c2_r)


def _sc_scatter(row, msg, coordc, n_pad):
    e, hdim = msg.shape
    ns = _NW // _NC
    ept = e // ns
    nch = ept // _EB
    npt = n_pad // ns
    mesh = plsc.VectorSubcoreMesh(core_axis_name="c", subcore_axis_name="s")

    @functools.partial(
        pl.kernel,
        mesh=mesh,
        out_type=(
            jax.ShapeDtypeStruct((n_pad, hdim), jnp.float32),
            jax.ShapeDtypeStruct((n_pad, hdim), jnp.float32),
        ),
        scratch_types=[
            pltpu.VMEM((_EB,), jnp.int32),
            pltpu.VMEM((_EB, hdim), jnp.float32),
            pltpu.VMEM((_EB, _XW), jnp.float32),
            pltpu.VMEM((8, hdim), jnp.float32),
            pltpu.VMEM_SHARED((n_pad, hdim), jnp.float32),
        ],
    )
    def k(row_h, msg_h, crd_h, agg_o, cagg_o,
          ri, mb, cb16, stg, acc):
        cid = lax.axis_index("c")
        sid = lax.axis_index("s")
        rbase = sid * npt

        for i in range(8):
            for kk in range(hdim // 16):
                stg[i, pl.ds(kk * 16, 16)] = jnp.zeros((16,), jnp.float32)

        def z3(t, c):
            pltpu.sync_copy(stg, acc.at[pl.ds(rbase + t * 8, 8)])
            return c
        lax.fori_loop(0, npt // 8, z3, 0)

        def z4(i, c):
            for kk in range(hdim // 16):
                mb[i, pl.ds(kk * 16, 16)] = jnp.zeros((16,), jnp.float32)
            return c
        lax.fori_loop(0, _EB, z4, 0)
        plsc.subcore_barrier()

        @pl.when(cid == 0)
        def _():
            def body(j, carry):
                base = sid * ept + j * _EB
                pltpu.sync_copy(row_h.at[pl.ds(base, _EB)], ri)
                pltpu.sync_copy(msg_h.at[pl.ds(base, _EB)], mb)
                pltpu.sync_copy(mb, acc.at[ri], add=True)
                return carry
            lax.fori_loop(0, nch, body, 0)

        @pl.when(cid == 1)
        def _():
            def body(j, carry):
                base = sid * ept + j * _EB
                pltpu.sync_copy(row_h.at[pl.ds(base, _EB)], ri)
                pltpu.sync_copy(crd_h.at[pl.ds(base, _EB)], cb16)

                def expand(i, c):
                    mb[i, pl.ds(0, _XW)] = cb16[i]
                    return c
                lax.fori_loop(0, _EB, expand, 0)
                pltpu.sync_copy(mb, acc.at[ri], add=True)
                return carry
            lax.fori_loop(0, nch, body, 0)

        plsc.subcore_barrier()

        def w1(t, c):
            pltpu.sync_copy(acc.at[pl.ds(rbase + t * 8, 8)], stg)

            @pl.when(cid == 0)
            def _():
                pltpu.sync_copy(stg, agg_o.at[pl.ds(rbase + t * 8, 8)])

            @pl.when(cid == 1)
            def _():
                pltpu.sync_copy(stg, cagg_o.at[pl.ds(rbase + t * 8, 8)])
            return c
        lax.fori_loop(0, npt // 8, w1, 0)

    return k(row, msg, coordc)


def _node_update(h, agg, cagg, xp, wn1h, wn1a, b_n1, w_n2,
                 b_n2, gamma, beta, inv_scale):
    n, d = h.shape
    hdim = wn1h.shape[1]
    bn = 1000

    def body(h_ref, a_ref, c_ref, xp_ref, w1h_ref, w1a_ref,
             b1_ref, w2_ref, b2_ref, g_ref, bt_ref, hout_ref, xout_ref):
        hb = h_ref[...]
        t = (jnp.dot(hb, w1h_ref[...], preferred_element_type=jnp.float32)
             + jnp.dot(a_ref[...], w1a_ref[...],
                       preferred_element_type=jnp.float32)
             + b1_ref[...])
        t = _silu(t)
        hn = jnp.dot(t, w2_ref[...], preferred_element_type=jnp.float32) + b2_ref[...]
        y = hb + hn
        mu = jnp.mean(y, axis=1, keepdims=True)
        yc = y - mu
        var = jnp.mean(yc * yc, axis=1, keepdims=True)
        hout_ref[...] = yc * jax.lax.rsqrt(var + 1e-05) * g_ref[...] + bt_ref[...]
        xout_ref[...] = xp_ref[...] + c_ref[:, :_XW] * inv_scale

    full = lambda s: pl.BlockSpec(s, lambda i: (0, 0))
    return pl.pallas_call(
        body,
        grid=(n // bn,),
        in_specs=[
            pl.BlockSpec((bn, d), lambda i: (i, 0)),
            pl.BlockSpec((bn, hdim), lambda i: (i, 0)),
            pl.BlockSpec((bn, hdim), lambda i: (i, 0)),
            pl.BlockSpec((bn, _XW), lambda i: (i, 0)),
            full((d, hdim)), full((hdim, hdim)), full((1, hdim)),
            full((hdim, d)), full((1, d)), full((1, d)), full((1, d)),
        ],
        out_specs=[
            pl.BlockSpec((bn, d), lambda i: (i, 0)),
            pl.BlockSpec((bn, _XW), lambda i: (i, 0)),
        ],
        out_shape=[
            jax.ShapeDtypeStruct((n, d), jnp.float32),
            jax.ShapeDtypeStruct((n, _XW), jnp.float32),
        ],
    )(h, agg, cagg, xp, wn1h, wn1a, b_n1, w_n2, b_n2,
      gamma, beta)


def kernel(h, x, edge_idx, edge_attr, W_e1, b_e1, W_e2, b_e2, W_att, b_att,
           W_n1, b_n1, W_n2, b_n2, W_c1, b_c1, W_c2, gamma, beta):
    n, d = h.shape
    e = edge_idx.shape[1]
    hdim = W_e2.shape[1]

    row = edge_idx[0].astype(jnp.int32)
    col = edge_idx[1].astype(jnp.int32)
    n_pad = ((n + 127) // 128) * 128
    xp = jnp.pad(x, ((0, n_pad - n), (0, 128 - x.shape[1])))
    xp16 = xp[:n, :_XW]

    w1a = W_e1[:d]
    w1b = W_e1[d:2 * d]
    w_d = W_e1[2 * d:2 * d + 1]
    w_ea = W_e1[2 * d + 1:]
    w_att_r = W_att.T
    w_c2_r = W_c2.T
    wn1h = W_n1[:d]
    wn1a = W_n1[d:]

    a_tab, b_tab = _precompute_ab(h, w1a, w1b)
    arow, bcol, dvec = _sc_gather(row, col, a_tab, b_tab, xp)
    msg, coordc = _edge_mlp(
        arow, bcol, dvec, edge_attr, w_ea, w_d, b_e1.reshape(1, -1),
        W_e2, b_e2.reshape(1, -1), w_att_r, b_att.reshape(1, 1), W_c1,
        b_c1.reshape(1, -1), w_c2_r)

    agg_p, cagg_p = _sc_scatter(row, msg, coordc, n_pad)

    inv_scale = 1.0 / (e / n + _EPS)
    h_out, x_out_p = _node_update(
        h, agg_p[:n], cagg_p[:n], xp16, wn1h, wn1a,
        b_n1.reshape(1, -1), W_n2, b_n2.reshape(1, -1), gamma.reshape(1, -1),
        beta.reshape(1, -1), inv_scale)
    return (h_out, x_out_p[:, :x.shape[1]])


# speedup vs baseline: 2.5960x; 1.0168x over previous
import functools

import jax
import jax.numpy as jnp
from jax import lax
from jax.experimental import pallas as pl
from jax.experimental.pallas import tpu as pltpu
from jax.experimental.pallas import tpu_sc as plsc

_EPS = 1e-08
_XW = 16
_EB = 80
_NW = 32
_NC = 2


def _silu(v):
    return v * jax.nn.sigmoid(v)


def _precompute_ab(h, w1a, w1b):
    n, d = h.shape
    bn = 1000
    def body(h_ref, wa_ref, wb_ref, a_ref, b_ref):
        hb = h_ref[...]
        a_ref[...] = jnp.dot(hb, wa_ref[...], preferred_element_type=jnp.float32)
        b_ref[...] = jnp.dot(hb, wb_ref[...], preferred_element_type=jnp.float32)
    return pl.pallas_call(
        body,
        grid=(n // bn,),
        in_specs=[
            pl.BlockSpec((bn, d), lambda i: (i, 0)),
            pl.BlockSpec(w1a.shape, lambda i: (0, 0)),
            pl.BlockSpec(w1b.shape, lambda i: (0, 0)),
        ],
        out_specs=[pl.BlockSpec((bn, w1a.shape[1]), lambda i: (i, 0))] * 2,
        out_shape=[jax.ShapeDtypeStruct((n, w1a.shape[1]), jnp.float32)] * 2,
    )(h, w1a, w1b)


def _sc_gather(row, col, a_tab, b_tab, x_tab):
    e = row.shape[0]
    n, hdim = a_tab.shape
    xw = x_tab.shape[1]
    ept = e // _NW
    nch = ept // _EB
    mesh = plsc.VectorSubcoreMesh(core_axis_name="c", subcore_axis_name="s")

    @functools.partial(
        pl.kernel,
        mesh=mesh,
        out_type=(
            jax.ShapeDtypeStruct((e, hdim), jnp.float32),
            jax.ShapeDtypeStruct((e, _XW), jnp.float32),
        ),
        scratch_types=[
            pltpu.VMEM((_EB,), jnp.int32),
            pltpu.VMEM((_EB,), jnp.int32),
            pltpu.VMEM((_EB, hdim), jnp.float32),
            pltpu.VMEM((_EB, hdim), jnp.float32),
            pltpu.VMEM((_EB, xw), jnp.float32),
            pltpu.VMEM((_EB, xw), jnp.float32),
            pltpu.VMEM((_EB, _XW), jnp.float32),
            pltpu.SemaphoreType.DMA,
        ],
    )
    def k(row_h, col_h, a_h, b_h, x_h, s_o, d_o,
          ri, ci, ab, bb, xrb, xcb, df, sem):
        wid = lax.axis_index("s") * _NC + lax.axis_index("c")
        base0 = wid * ept

        def body(j, carry):
            base = base0 + j * _EB
            pltpu.sync_copy(row_h.at[pl.ds(base, _EB)], ri)
            pltpu.sync_copy(col_h.at[pl.ds(base, _EB)], ci)
            c1 = pltpu.async_copy(a_h.at[ri], ab, sem)
            c2 = pltpu.async_copy(b_h.at[ci], bb, sem)
            c3 = pltpu.async_copy(x_h.at[ri], xrb, sem)
            c4 = pltpu.async_copy(x_h.at[ci], xcb, sem)
            c1.wait(); c2.wait(); c3.wait(); c4.wait()

            def srow(i, c):
                for kk in range(hdim // 16):
                    sl = pl.ds(kk * 16, 16)
                    ab[i, sl] = ab[i, sl] + bb[i, sl]
                df[i] = xrb[i, pl.ds(0, _XW)] - xcb[i, pl.ds(0, _XW)]
                return c
            lax.fori_loop(0, _EB, srow, 0)

            pltpu.sync_copy(ab, s_o.at[pl.ds(base, _EB)])
            pltpu.sync_copy(df, d_o.at[pl.ds(base, _EB)])
            return carry

        lax.fori_loop(0, nch, body, 0)

    return k(row, col, a_tab, b_tab, x_tab)


def _edge_mlp(svec, dvec, eattr, w_ea, w_d, b_e1, w_e2, b_e2,
              w_att_r, b_att, w_c1, b_c1, w_c2_r):
    e, hdim = svec.shape
    ed = eattr.shape[1]
    be = 512

    def body(s_ref, d_ref, ea_ref, wea_ref, wd_ref, be1_ref,
             we2_ref, be2_ref, watt_ref, batt_ref, wc1_ref, bc1_ref, wc2_ref,
             msg_ref, crd_ref):
        d = d_ref[...]
        dist_sq = jnp.sum(d * d, axis=1, keepdims=True)
        pre1 = (s_ref[...] + dist_sq * wd_ref[...]
                + jnp.dot(ea_ref[...], wea_ref[...],
                          preferred_element_type=jnp.float32) + be1_ref[...])
        t = _silu(pre1)
        msg0 = _silu(jnp.dot(t, we2_ref[...],
                             preferred_element_type=jnp.float32) + be2_ref[...])
        att = jax.nn.sigmoid(
            jnp.sum(msg0 * watt_ref[...], axis=1, keepdims=True) + batt_ref[...])
        msg = msg0 * att
        u = _silu(jnp.dot(msg, wc1_ref[...],
                          preferred_element_type=jnp.float32) + bc1_ref[...])
        cw = jnp.tanh(jnp.sum(u * wc2_ref[...], axis=1, keepdims=True))
        unit = d * jax.lax.rsqrt(dist_sq + _EPS)
        msg_ref[...] = msg
        crd_ref[...] = cw * unit

    full = lambda s: pl.BlockSpec(s, lambda i: (0, 0))
    return pl.pallas_call(
        body,
        grid=(e // be,),
        in_specs=[
            pl.BlockSpec((be, hdim), lambda i: (i, 0)),
            pl.BlockSpec((be, _XW), lambda i: (i, 0)),
            pl.BlockSpec((be, ed), lambda i: (i, 0)),
            full((ed, hdim)), full((1, hdim)), full((1, hdim)),
            full((hdim, hdim)), full((1, hdim)),
            full((1, hdim)), full((1, 1)),
            full((hdim, hdim)), full((1, hdim)), full((1, hdim)),
        ],
        out_specs=[
            pl.BlockSpec((be, hdim), lambda i: (i, 0)),
            pl.BlockSpec((be, _XW), lambda i: (i, 0)),
        ],
        out_shape=[
            jax.ShapeDtypeStruct((e, hdim), jnp.float32),
            jax.ShapeDtypeStruct((e, _XW), jnp.float32),
        ],
    )(svec, dvec, eattr, w_ea, w_d, b_e1, w_e2, b_e2,
      w_att_r, b_att, w_c1, b_c1, w_c2_r)


def _sc_scatter(row, msg, coordc, n_pad):
    e, hdim = msg.shape
    ns = _NW // _NC
    ept = e // ns
    nch = ept // _EB
    npt = n_pad // ns
    mesh = plsc.VectorSubcoreMesh(core_axis_name="c", subcore_axis_name="s")

    @functools.partial(
        pl.kernel,
        mesh=mesh,
        out_type=(
            jax.ShapeDtypeStruct((n_pad, hdim), jnp.float32),
            jax.ShapeDtypeStruct((n_pad, hdim), jnp.float32),
        ),
        scratch_types=[
            pltpu.VMEM((_EB,), jnp.int32),
            pltpu.VMEM((_EB, hdim), jnp.float32),
            pltpu.VMEM((_EB, _XW), jnp.float32),
            pltpu.VMEM((8, hdim), jnp.float32),
            pltpu.VMEM_SHARED((n_pad, hdim), jnp.float32),
        ],
    )
    def k(row_h, msg_h, crd_h, agg_o, cagg_o,
          ri, mb, cb16, stg, acc):
        cid = lax.axis_index("c")
        sid = lax.axis_index("s")
        rbase = sid * npt

        for i in range(8):
            for kk in range(hdim // 16):
                stg[i, pl.ds(kk * 16, 16)] = jnp.zeros((16,), jnp.float32)

        def z3(t, c):
            pltpu.sync_copy(stg, acc.at[pl.ds(rbase + t * 8, 8)])
            return c
        lax.fori_loop(0, npt // 8, z3, 0)

        def z4(i, c):
            for kk in range(hdim // 16):
                mb[i, pl.ds(kk * 16, 16)] = jnp.zeros((16,), jnp.float32)
            return c
        lax.fori_loop(0, _EB, z4, 0)
        plsc.subcore_barrier()

        @pl.when(cid == 0)
        def _():
            def body(j, carry):
                base = sid * ept + j * _EB
                pltpu.sync_copy(row_h.at[pl.ds(base, _EB)], ri)
                pltpu.sync_copy(msg_h.at[pl.ds(base, _EB)], mb)
                pltpu.sync_copy(mb, acc.at[ri], add=True)
                return carry
            lax.fori_loop(0, nch, body, 0)

        @pl.when(cid == 1)
        def _():
            def body(j, carry):
                base = sid * ept + j * _EB
                pltpu.sync_copy(row_h.at[pl.ds(base, _EB)], ri)
                pltpu.sync_copy(crd_h.at[pl.ds(base, _EB)], cb16)

                def expand(i, c):
                    mb[i, pl.ds(0, _XW)] = cb16[i]
                    return c
                lax.fori_loop(0, _EB, expand, 0)
                pltpu.sync_copy(mb, acc.at[ri], add=True)
                return carry
            lax.fori_loop(0, nch, body, 0)

        plsc.subcore_barrier()

        def w1(t, c):
            pltpu.sync_copy(acc.at[pl.ds(rbase + t * 8, 8)], stg)

            @pl.when(cid == 0)
            def _():
                pltpu.sync_copy(stg, agg_o.at[pl.ds(rbase + t * 8, 8)])

            @pl.when(cid == 1)
            def _():
                pltpu.sync_copy(stg, cagg_o.at[pl.ds(rbase + t * 8, 8)])
            return c
        lax.fori_loop(0, npt // 8, w1, 0)

    return k(row, msg, coordc)


def _node_update(h, agg, cagg, xp, wn1h, wn1a, b_n1, w_n2,
                 b_n2, gamma, beta, inv_scale):
    n, d = h.shape
    hdim = wn1h.shape[1]
    bn = 1000

    def body(h_ref, a_ref, c_ref, xp_ref, w1h_ref, w1a_ref,
             b1_ref, w2_ref, b2_ref, g_ref, bt_ref, hout_ref, xout_ref):
        hb = h_ref[...]
        t = (jnp.dot(hb, w1h_ref[...], preferred_element_type=jnp.float32)
             + jnp.dot(a_ref[...], w1a_ref[...],
                       preferred_element_type=jnp.float32)
             + b1_ref[...])
        t = _silu(t)
        hn = jnp.dot(t, w2_ref[...], preferred_element_type=jnp.float32) + b2_ref[...]
        y = hb + hn
        mu = jnp.mean(y, axis=1, keepdims=True)
        yc = y - mu
        var = jnp.mean(yc * yc, axis=1, keepdims=True)
        hout_ref[...] = yc * jax.lax.rsqrt(var + 1e-05) * g_ref[...] + bt_ref[...]
        xout_ref[...] = xp_ref[...] + c_ref[:, :_XW] * inv_scale

    full = lambda s: pl.BlockSpec(s, lambda i: (0, 0))
    return pl.pallas_call(
        body,
        grid=(n // bn,),
        in_specs=[
            pl.BlockSpec((bn, d), lambda i: (i, 0)),
            pl.BlockSpec((bn, hdim), lambda i: (i, 0)),
            pl.BlockSpec((bn, hdim), lambda i: (i, 0)),
            pl.BlockSpec((bn, _XW), lambda i: (i, 0)),
            full((d, hdim)), full((hdim, hdim)), full((1, hdim)),
            full((hdim, d)), full((1, d)), full((1, d)), full((1, d)),
        ],
        out_specs=[
            pl.BlockSpec((bn, d), lambda i: (i, 0)),
            pl.BlockSpec((bn, _XW), lambda i: (i, 0)),
        ],
        out_shape=[
            jax.ShapeDtypeStruct((n, d), jnp.float32),
            jax.ShapeDtypeStruct((n, _XW), jnp.float32),
        ],
    )(h, agg, cagg, xp, wn1h, wn1a, b_n1, w_n2, b_n2,
      gamma, beta)


def kernel(h, x, edge_idx, edge_attr, W_e1, b_e1, W_e2, b_e2, W_att, b_att,
           W_n1, b_n1, W_n2, b_n2, W_c1, b_c1, W_c2, gamma, beta):
    n, d = h.shape
    e = edge_idx.shape[1]
    hdim = W_e2.shape[1]

    row = edge_idx[0].astype(jnp.int32)
    col = edge_idx[1].astype(jnp.int32)
    n_pad = ((n + 127) // 128) * 128
    xp = jnp.pad(x, ((0, n_pad - n), (0, 128 - x.shape[1])))
    xp16 = xp[:n, :_XW]

    w1a = W_e1[:d]
    w1b = W_e1[d:2 * d]
    w_d = W_e1[2 * d:2 * d + 1]
    w_ea = W_e1[2 * d + 1:]
    w_att_r = W_att.T
    w_c2_r = W_c2.T
    wn1h = W_n1[:d]
    wn1a = W_n1[d:]

    a_tab, b_tab = _precompute_ab(h, w1a, w1b)
    svec, dvec = _sc_gather(row, col, a_tab, b_tab, xp)
    msg, coordc = _edge_mlp(
        svec, dvec, edge_attr, w_ea, w_d, b_e1.reshape(1, -1),
        W_e2, b_e2.reshape(1, -1), w_att_r, b_att.reshape(1, 1), W_c1,
        b_c1.reshape(1, -1), w_c2_r)

    agg_p, cagg_p = _sc_scatter(row, msg, coordc, n_pad)

    inv_scale = 1.0 / (e / n + _EPS)
    h_out, x_out_p = _node_update(
        h, agg_p[:n], cagg_p[:n], xp16, wn1h, wn1a,
        b_n1.reshape(1, -1), W_n2, b_n2.reshape(1, -1), gamma.reshape(1, -1),
        beta.reshape(1, -1), inv_scale)
    return (h_out, x_out_p[:, :x.shape[1]])


# speedup vs baseline: 3.4062x; 1.3121x over previous
import functools

import jax
import jax.numpy as jnp
from jax import lax
from jax.experimental import pallas as pl
from jax.experimental.pallas import tpu as pltpu
from jax.experimental.pallas import tpu_sc as plsc

_EPS = 1e-08
_XW = 16
_EB = 80
_NW = 32
_NC = 2


def _silu(v):
    return v * jax.nn.sigmoid(v)


def _precompute_ab(h, w1a, w1b):
    n, d = h.shape
    bn = 1000
    def body(h_ref, wa_ref, wb_ref, a_ref, b_ref):
        hb = h_ref[...]
        a_ref[...] = jnp.dot(hb, wa_ref[...], preferred_element_type=jnp.float32)
        b_ref[...] = jnp.dot(hb, wb_ref[...], preferred_element_type=jnp.float32)
    return pl.pallas_call(
        body,
        grid=(n // bn,),
        in_specs=[
            pl.BlockSpec((bn, d), lambda i: (i, 0)),
            pl.BlockSpec(w1a.shape, lambda i: (0, 0)),
            pl.BlockSpec(w1b.shape, lambda i: (0, 0)),
        ],
        out_specs=[pl.BlockSpec((bn, w1a.shape[1]), lambda i: (i, 0))] * 2,
        out_shape=[jax.ShapeDtypeStruct((n, w1a.shape[1]), jnp.float32)] * 2,
    )(h, w1a, w1b)


def _sc_gather(row, col, a_tab, b_tab, x_tab):
    e = row.shape[0]
    n, hdim = a_tab.shape
    xw = x_tab.shape[1]
    ept = e // _NW
    nch = ept // _EB
    mesh = plsc.VectorSubcoreMesh(core_axis_name="c", subcore_axis_name="s")

    @functools.partial(
        pl.kernel,
        mesh=mesh,
        out_type=(
            jax.ShapeDtypeStruct((e, hdim), jnp.float32),
            jax.ShapeDtypeStruct((e, _XW), jnp.float32),
        ),
        scratch_types=[
            pltpu.VMEM((_EB,), jnp.int32),
            pltpu.VMEM((_EB,), jnp.int32),
            pltpu.VMEM((_EB, hdim), jnp.float32),
            pltpu.VMEM((_EB, hdim), jnp.float32),
            pltpu.VMEM((_EB, xw), jnp.float32),
            pltpu.VMEM((_EB, xw), jnp.float32),
            pltpu.VMEM((_EB, _XW), jnp.float32),
            pltpu.SemaphoreType.DMA,
        ],
    )
    def k(row_h, col_h, a_h, b_h, x_h, s_o, d_o,
          ri, ci, ab, bb, xrb, xcb, df, sem):
        wid = lax.axis_index("s") * _NC + lax.axis_index("c")
        base0 = wid * ept

        def body(j, carry):
            base = base0 + j * _EB
            pltpu.sync_copy(row_h.at[pl.ds(base, _EB)], ri)
            pltpu.sync_copy(col_h.at[pl.ds(base, _EB)], ci)
            c1 = pltpu.async_copy(a_h.at[ri], ab, sem)
            c2 = pltpu.async_copy(b_h.at[ci], bb, sem)
            c3 = pltpu.async_copy(x_h.at[ri], xrb, sem)
            c4 = pltpu.async_copy(x_h.at[ci], xcb, sem)
            c1.wait(); c2.wait(); c3.wait(); c4.wait()

            def srow(i, c):
                for kk in range(hdim // 16):
                    sl = pl.ds(kk * 16, 16)
                    ab[i, sl] = ab[i, sl] + bb[i, sl]
                df[i] = xrb[i, pl.ds(0, _XW)] - xcb[i, pl.ds(0, _XW)]
                return c
            lax.fori_loop(0, _EB, srow, 0)

            pltpu.sync_copy(ab, s_o.at[pl.ds(base, _EB)])
            pltpu.sync_copy(df, d_o.at[pl.ds(base, _EB)])
            return carry

        lax.fori_loop(0, nch, body, 0)

    return k(row, col, a_tab, b_tab, x_tab)


def _edge_mlp(svec, dvec, eattr, w_ea, w_d, b_e1, w_e2, b_e2,
              w_att_r, b_att, w_c1, b_c1, w_c2_r):
    e, hdim = svec.shape
    ed = eattr.shape[1]
    be = 512

    def body(s_ref, d_ref, ea_ref, wea_ref, wd_ref, be1_ref,
             we2_ref, be2_ref, watt_ref, batt_ref, wc1_ref, bc1_ref, wc2_ref,
             msg_ref, crd_ref):
        d = d_ref[...]
        dist_sq = jnp.sum(d * d, axis=1, keepdims=True)
        pre1 = (s_ref[...] + dist_sq * wd_ref[...]
                + jnp.dot(ea_ref[...], wea_ref[...],
                          preferred_element_type=jnp.float32) + be1_ref[...])
        t = _silu(pre1)
        msg0 = _silu(jnp.dot(t, we2_ref[...],
                             preferred_element_type=jnp.float32) + be2_ref[...])
        att = jax.nn.sigmoid(
            jnp.sum(msg0 * watt_ref[...], axis=1, keepdims=True) + batt_ref[...])
        msg = msg0 * att
        u = _silu(jnp.dot(msg, wc1_ref[...],
                          preferred_element_type=jnp.float32) + bc1_ref[...])
        cw = jnp.tanh(jnp.sum(u * wc2_ref[...], axis=1, keepdims=True))
        unit = d * jax.lax.rsqrt(dist_sq + _EPS)
        msg_ref[...] = msg
        crd_ref[...] = cw * unit

    full = lambda s: pl.BlockSpec(s, lambda i: (0, 0))
    return pl.pallas_call(
        body,
        grid=(e // be,),
        in_specs=[
            pl.BlockSpec((be, hdim), lambda i: (i, 0)),
            pl.BlockSpec((be, _XW), lambda i: (i, 0)),
            pl.BlockSpec((be, ed), lambda i: (i, 0)),
            full((ed, hdim)), full((1, hdim)), full((1, hdim)),
            full((hdim, hdim)), full((1, hdim)),
            full((1, hdim)), full((1, 1)),
            full((hdim, hdim)), full((1, hdim)), full((1, hdim)),
        ],
        out_specs=[
            pl.BlockSpec((be, hdim), lambda i: (i, 0)),
            pl.BlockSpec((be, _XW), lambda i: (i, 0)),
        ],
        out_shape=[
            jax.ShapeDtypeStruct((e, hdim), jnp.float32),
            jax.ShapeDtypeStruct((e, _XW), jnp.float32),
        ],
    )(svec, dvec, eattr, w_ea, w_d, b_e1, w_e2, b_e2,
      w_att_r, b_att, w_c1, b_c1, w_c2_r)


def _sc_scatter(row, msg, coordc, n_pad):
    e, hdim = msg.shape
    ns = _NW // _NC
    ept = e // ns
    nch = ept // _EB
    npt = n_pad // ns
    mesh = plsc.VectorSubcoreMesh(core_axis_name="c", subcore_axis_name="s")

    @functools.partial(
        pl.kernel,
        mesh=mesh,
        out_type=(
            jax.ShapeDtypeStruct((n_pad, hdim), jnp.float32),
            jax.ShapeDtypeStruct((n_pad, hdim), jnp.float32),
        ),
        scratch_types=[
            pltpu.VMEM((_EB,), jnp.int32),
            pltpu.VMEM((_EB, hdim), jnp.float32),
            pltpu.VMEM((_EB, _XW), jnp.float32),
            pltpu.VMEM((8, hdim), jnp.float32),
            pltpu.VMEM_SHARED((n_pad, hdim), jnp.float32),
        ],
    )
    def k(row_h, msg_h, crd_h, agg_o, cagg_o,
          ri, mb, cb16, stg, acc):
        cid = lax.axis_index("c")
        sid = lax.axis_index("s")
        rbase = sid * npt

        for i in range(8):
            for kk in range(hdim // 16):
                stg[i, pl.ds(kk * 16, 16)] = jnp.zeros((16,), jnp.float32)

        def z3(t, c):
            pltpu.sync_copy(stg, acc.at[pl.ds(rbase + t * 8, 8)])
            return c
        lax.fori_loop(0, npt // 8, z3, 0)

        def z4(i, c):
            for kk in range(hdim // 16):
                mb[i, pl.ds(kk * 16, 16)] = jnp.zeros((16,), jnp.float32)
            return c
        lax.fori_loop(0, _EB, z4, 0)
        plsc.subcore_barrier()

        @pl.when(cid == 0)
        def _():
            def body(j, carry):
                base = sid * ept + j * _EB
                pltpu.sync_copy(row_h.at[pl.ds(base, _EB)], ri)
                pltpu.sync_copy(msg_h.at[pl.ds(base, _EB)], mb)
                pltpu.sync_copy(mb, acc.at[ri], add=True)
                return carry
            lax.fori_loop(0, nch, body, 0)

        @pl.when(cid == 1)
        def _():
            def body(j, carry):
                base = sid * ept + j * _EB
                pltpu.sync_copy(row_h.at[pl.ds(base, _EB)], ri)
                pltpu.sync_copy(crd_h.at[pl.ds(base, _EB)], cb16)

                def expand(i, c):
                    mb[i, pl.ds(0, _XW)] = cb16[i]
                    return c
                lax.fori_loop(0, _EB, expand, 0)
                pltpu.sync_copy(mb, acc.at[ri], add=True)
                return carry
            lax.fori_loop(0, nch, body, 0)

        plsc.subcore_barrier()

        def w1(t, c):
            pltpu.sync_copy(acc.at[pl.ds(rbase + t * 8, 8)], stg)

            @pl.when(cid == 0)
            def _():
                pltpu.sync_copy(stg, agg_o.at[pl.ds(rbase + t * 8, 8)])

            @pl.when(cid == 1)
            def _():
                pltpu.sync_copy(stg, cagg_o.at[pl.ds(rbase + t * 8, 8)])
            return c
        lax.fori_loop(0, npt // 8, w1, 0)

    return k(row, msg, coordc)


def _node_update(h, agg_a, agg_b, cagg_a, cagg_b, xp, wn1h, wn1a, b_n1, w_n2,
                 b_n2, gamma, beta, inv_scale):
    n, d = h.shape
    hdim = wn1h.shape[1]
    bn = 1000

    def body(h_ref, aa_ref, ab_ref, ca_ref, cb_ref, xp_ref, w1h_ref, w1a_ref,
             b1_ref, w2_ref, b2_ref, g_ref, bt_ref, hout_ref, xout_ref):
        hb = h_ref[...]
        agg = aa_ref[...] + ab_ref[...]
        t = (jnp.dot(hb, w1h_ref[...], preferred_element_type=jnp.float32)
             + jnp.dot(agg, w1a_ref[...],
                       preferred_element_type=jnp.float32)
             + b1_ref[...])
        t = _silu(t)
        hn = jnp.dot(t, w2_ref[...], preferred_element_type=jnp.float32) + b2_ref[...]
        y = hb + hn
        mu = jnp.mean(y, axis=1, keepdims=True)
        yc = y - mu
        var = jnp.mean(yc * yc, axis=1, keepdims=True)
        hout_ref[...] = yc * jax.lax.rsqrt(var + 1e-05) * g_ref[...] + bt_ref[...]
        xout_ref[...] = (xp_ref[...]
                         + (ca_ref[:, :_XW] + cb_ref[:, :_XW]) * inv_scale)

    full = lambda s: pl.BlockSpec(s, lambda i: (0, 0))
    return pl.pallas_call(
        body,
        grid=(n // bn,),
        in_specs=[
            pl.BlockSpec((bn, d), lambda i: (i, 0)),
            pl.BlockSpec((bn, hdim), lambda i: (i, 0)),
            pl.BlockSpec((bn, hdim), lambda i: (i, 0)),
            pl.BlockSpec((bn, hdim), lambda i: (i, 0)),
            pl.BlockSpec((bn, hdim), lambda i: (i, 0)),
            pl.BlockSpec((bn, _XW), lambda i: (i, 0)),
            full((d, hdim)), full((hdim, hdim)), full((1, hdim)),
            full((hdim, d)), full((1, d)), full((1, d)), full((1, d)),
        ],
        out_specs=[
            pl.BlockSpec((bn, d), lambda i: (i, 0)),
            pl.BlockSpec((bn, _XW), lambda i: (i, 0)),
        ],
        out_shape=[
            jax.ShapeDtypeStruct((n, d), jnp.float32),
            jax.ShapeDtypeStruct((n, _XW), jnp.float32),
        ],
    )(h, agg_a, agg_b, cagg_a, cagg_b, xp, wn1h, wn1a, b_n1, w_n2, b_n2,
      gamma, beta)


def kernel(h, x, edge_idx, edge_attr, W_e1, b_e1, W_e2, b_e2, W_att, b_att,
           W_n1, b_n1, W_n2, b_n2, W_c1, b_c1, W_c2, gamma, beta):
    n, d = h.shape
    e = edge_idx.shape[1]
    hdim = W_e2.shape[1]

    row = edge_idx[0].astype(jnp.int32)
    col = edge_idx[1].astype(jnp.int32)
    n_pad = ((n + 127) // 128) * 128
    xp = jnp.pad(x, ((0, n_pad - n), (0, 128 - x.shape[1])))
    xp16 = xp[:n, :_XW]

    w1a = W_e1[:d]
    w1b = W_e1[d:2 * d]
    w_d = W_e1[2 * d:2 * d + 1]
    w_ea = W_e1[2 * d + 1:]
    w_att_r = W_att.T
    w_c2_r = W_c2.T
    wn1h = W_n1[:d]
    wn1a = W_n1[d:]

    a_tab, b_tab = _precompute_ab(h, w1a, w1b)

    grp = _NW * _EB
    e0 = ((e // grp) // 2) * grp
    mlp_args = (edge_attr, w_ea, w_d, b_e1.reshape(1, -1),
                W_e2, b_e2.reshape(1, -1), w_att_r, b_att.reshape(1, 1), W_c1,
                b_c1.reshape(1, -1), w_c2_r)

    s0, d0 = _sc_gather(row[:e0], col[:e0], a_tab, b_tab, xp)
    s1, d1 = _sc_gather(row[e0:], col[e0:], a_tab, b_tab, xp)
    msg0, crd0 = _edge_mlp(s0, d0, edge_attr[:e0], *mlp_args[1:])
    msg1, crd1 = _edge_mlp(s1, d1, edge_attr[e0:], *mlp_args[1:])
    agg_a, cagg_a = _sc_scatter(row[:e0], msg0, crd0, n_pad)
    agg_b, cagg_b = _sc_scatter(row[e0:], msg1, crd1, n_pad)

    inv_scale = 1.0 / (e / n + _EPS)
    h_out, x_out_p = _node_update(
        h, agg_a[:n], agg_b[:n], cagg_a[:n], cagg_b[:n], xp16, wn1h, wn1a,
        b_n1.reshape(1, -1), W_n2, b_n2.reshape(1, -1), gamma.reshape(1, -1),
        beta.reshape(1, -1), inv_scale)
    return (h_out, x_out_p[:, :x.shape[1]])


# speedup vs baseline: 3.6159x; 1.0616x over previous
import functools

import jax
import jax.numpy as jnp
from jax import lax
from jax.experimental import pallas as pl
from jax.experimental.pallas import tpu as pltpu
from jax.experimental.pallas import tpu_sc as plsc

_EPS = 1e-08
_XW = 16
_EB = 80
_NW = 32
_NC = 2


def _silu(v):
    return v * jax.nn.sigmoid(v)


def _precompute_ab(h, w1a, w1b):
    n, d = h.shape
    bn = 1000
    def body(h_ref, wa_ref, wb_ref, a_ref, b_ref):
        hb = h_ref[...]
        a_ref[...] = jnp.dot(hb, wa_ref[...], preferred_element_type=jnp.float32)
        b_ref[...] = jnp.dot(hb, wb_ref[...], preferred_element_type=jnp.float32)
    return pl.pallas_call(
        body,
        grid=(n // bn,),
        in_specs=[
            pl.BlockSpec((bn, d), lambda i: (i, 0)),
            pl.BlockSpec(w1a.shape, lambda i: (0, 0)),
            pl.BlockSpec(w1b.shape, lambda i: (0, 0)),
        ],
        out_specs=[pl.BlockSpec((bn, w1a.shape[1]), lambda i: (i, 0))] * 2,
        out_shape=[jax.ShapeDtypeStruct((n, w1a.shape[1]), jnp.float32)] * 2,
    )(h, w1a, w1b)


def _sc_gather(row, col, a_tab, b_tab, x_tab):
    e = row.shape[0]
    n, hdim = a_tab.shape
    xw = x_tab.shape[1]
    ept = e // _NW
    nch = ept // _EB
    mesh = plsc.VectorSubcoreMesh(core_axis_name="c", subcore_axis_name="s")

    @functools.partial(
        pl.kernel,
        mesh=mesh,
        out_type=(
            jax.ShapeDtypeStruct((e, hdim), jnp.float32),
            jax.ShapeDtypeStruct((e, _XW), jnp.float32),
        ),
        scratch_types=[
            pltpu.VMEM((_EB,), jnp.int32),
            pltpu.VMEM((_EB,), jnp.int32),
            pltpu.VMEM((_EB, hdim), jnp.float32),
            pltpu.VMEM((_EB, hdim), jnp.float32),
            pltpu.VMEM((_EB, xw), jnp.float32),
            pltpu.VMEM((_EB, xw), jnp.float32),
            pltpu.VMEM((_EB, _XW), jnp.float32),
            pltpu.VMEM((_EB,), jnp.int32),
            pltpu.VMEM((_EB,), jnp.int32),
            pltpu.VMEM((_EB, hdim), jnp.float32),
            pltpu.VMEM((_EB, hdim), jnp.float32),
            pltpu.VMEM((_EB, xw), jnp.float32),
            pltpu.VMEM((_EB, xw), jnp.float32),
            pltpu.VMEM((_EB, _XW), jnp.float32),
            pltpu.SemaphoreType.DMA,
            pltpu.SemaphoreType.DMA,
            pltpu.SemaphoreType.DMA,
        ],
    )
    def k(row_h, col_h, a_h, b_h, x_h, s_o, d_o,
          ri0, ci0, ab0, bb0, xr0, xc0, df0,
          ri1, ci1, ab1, bb1, xr1, xc1, df1,
          semA, semB, semW):
        wid = lax.axis_index("s") * _NC + lax.axis_index("c")
        base0 = wid * ept

        def addrows(ab, bb, xr, xc, df):
            def srow(i, c):
                for kk in range(hdim // 16):
                    sl = pl.ds(kk * 16, 16)
                    ab[i, sl] = ab[i, sl] + bb[i, sl]
                df[i] = xr[i, pl.ds(0, _XW)] - xc[i, pl.ds(0, _XW)]
                return c
            lax.fori_loop(0, _EB, srow, 0)

        def fire(b, ri, ci, ab, bb, xr, xc, sem):
            pltpu.sync_copy(row_h.at[pl.ds(b, _EB)], ri)
            pltpu.sync_copy(col_h.at[pl.ds(b, _EB)], ci)
            return (pltpu.async_copy(a_h.at[ri], ab, sem),
                    pltpu.async_copy(b_h.at[ci], bb, sem),
                    pltpu.async_copy(x_h.at[ri], xr, sem),
                    pltpu.async_copy(x_h.at[ci], xc, sem))

        def body(jj, carry):
            b0 = base0 + (2 * jj) * _EB
            b1 = b0 + _EB
            g0 = fire(b0, ri0, ci0, ab0, bb0, xr0, xc0, semA)
            g1 = fire(b1, ri1, ci1, ab1, bb1, xr1, xc1, semB)
            for c in g0:
                c.wait()
            addrows(ab0, bb0, xr0, xc0, df0)
            w0 = (pltpu.async_copy(ab0, s_o.at[pl.ds(b0, _EB)], semW),
                  pltpu.async_copy(df0, d_o.at[pl.ds(b0, _EB)], semW))
            for c in g1:
                c.wait()
            addrows(ab1, bb1, xr1, xc1, df1)
            w1 = (pltpu.async_copy(ab1, s_o.at[pl.ds(b1, _EB)], semW),
                  pltpu.async_copy(df1, d_o.at[pl.ds(b1, _EB)], semW))
            for c in w0 + w1:
                c.wait()
            return carry

        lax.fori_loop(0, nch // 2, body, 0)

        if nch % 2:
            bt = base0 + (nch - 1) * _EB
            gt = fire(bt, ri0, ci0, ab0, bb0, xr0, xc0, semA)
            for c in gt:
                c.wait()
            addrows(ab0, bb0, xr0, xc0, df0)
            pltpu.sync_copy(ab0, s_o.at[pl.ds(bt, _EB)])
            pltpu.sync_copy(df0, d_o.at[pl.ds(bt, _EB)])

    return k(row, col, a_tab, b_tab, x_tab)


def _edge_mlp(svec, dvec, eattr, w_ea, w_d, b_e1, w_e2, b_e2,
              w_att_r, b_att, w_c1, b_c1, w_c2_r):
    e, hdim = svec.shape
    ed = eattr.shape[1]
    be = 512

    def body(s_ref, d_ref, ea_ref, wea_ref, wd_ref, be1_ref,
             we2_ref, be2_ref, watt_ref, batt_ref, wc1_ref, bc1_ref, wc2_ref,
             msg_ref, crd_ref):
        d = d_ref[...]
        dist_sq = jnp.sum(d * d, axis=1, keepdims=True)
        pre1 = (s_ref[...] + dist_sq * wd_ref[...]
                + jnp.dot(ea_ref[...], wea_ref[...],
                          preferred_element_type=jnp.float32) + be1_ref[...])
        t = _silu(pre1)
        msg0 = _silu(jnp.dot(t, we2_ref[...],
                             preferred_element_type=jnp.float32) + be2_ref[...])
        att = jax.nn.sigmoid(
            jnp.sum(msg0 * watt_ref[...], axis=1, keepdims=True) + batt_ref[...])
        msg = msg0 * att
        u = _silu(jnp.dot(msg, wc1_ref[...],
                          preferred_element_type=jnp.float32) + bc1_ref[...])
        cw = jnp.tanh(jnp.sum(u * wc2_ref[...], axis=1, keepdims=True))
        unit = d * jax.lax.rsqrt(dist_sq + _EPS)
        msg_ref[...] = msg
        crd_ref[...] = cw * unit

    full = lambda s: pl.BlockSpec(s, lambda i: (0, 0))
    return pl.pallas_call(
        body,
        grid=(e // be,),
        in_specs=[
            pl.BlockSpec((be, hdim), lambda i: (i, 0)),
            pl.BlockSpec((be, _XW), lambda i: (i, 0)),
            pl.BlockSpec((be, ed), lambda i: (i, 0)),
            full((ed, hdim)), full((1, hdim)), full((1, hdim)),
            full((hdim, hdim)), full((1, hdim)),
            full((1, hdim)), full((1, 1)),
            full((hdim, hdim)), full((1, hdim)), full((1, hdim)),
        ],
        out_specs=[
            pl.BlockSpec((be, hdim), lambda i: (i, 0)),
            pl.BlockSpec((be, _XW), lambda i: (i, 0)),
        ],
        out_shape=[
            jax.ShapeDtypeStruct((e, hdim), jnp.float32),
            jax.ShapeDtypeStruct((e, _XW), jnp.float32),
        ],
    )(svec, dvec, eattr, w_ea, w_d, b_e1, w_e2, b_e2,
      w_att_r, b_att, w_c1, b_c1, w_c2_r)


def _sc_scatter(row, msg, coordc, n_pad):
    e, hdim = msg.shape
    ns = _NW // _NC
    ept = e // ns
    nch = ept // _EB
    npt = n_pad // ns
    mesh = plsc.VectorSubcoreMesh(core_axis_name="c", subcore_axis_name="s")

    @functools.partial(
        pl.kernel,
        mesh=mesh,
        out_type=(
            jax.ShapeDtypeStruct((n_pad, hdim), jnp.float32),
            jax.ShapeDtypeStruct((n_pad, hdim), jnp.float32),
        ),
        scratch_types=[
            pltpu.VMEM((_EB,), jnp.int32),
            pltpu.VMEM((_EB, hdim), jnp.float32),
            pltpu.VMEM((_EB, _XW), jnp.float32),
            pltpu.VMEM((8, hdim), jnp.float32),
            pltpu.VMEM_SHARED((n_pad, hdim), jnp.float32),
        ],
    )
    def k(row_h, msg_h, crd_h, agg_o, cagg_o,
          ri, mb, cb16, stg, acc):
        cid = lax.axis_index("c")
        sid = lax.axis_index("s")
        rbase = sid * npt

        for i in range(8):
            for kk in range(hdim // 16):
                stg[i, pl.ds(kk * 16, 16)] = jnp.zeros((16,), jnp.float32)

        def z3(t, c):
            pltpu.sync_copy(stg, acc.at[pl.ds(rbase + t * 8, 8)])
            return c
        lax.fori_loop(0, npt // 8, z3, 0)

        def z4(i, c):
            for kk in range(hdim // 16):
                mb[i, pl.ds(kk * 16, 16)] = jnp.zeros((16,), jnp.float32)
            return c
        lax.fori_loop(0, _EB, z4, 0)
        plsc.subcore_barrier()

        @pl.when(cid == 0)
        def _():
            def body(j, carry):
                base = sid * ept + j * _EB
                pltpu.sync_copy(row_h.at[pl.ds(base, _EB)], ri)
                pltpu.sync_copy(msg_h.at[pl.ds(base, _EB)], mb)
                pltpu.sync_copy(mb, acc.at[ri], add=True)
                return carry
            lax.fori_loop(0, nch, body, 0)

        @pl.when(cid == 1)
        def _():
            def body(j, carry):
                base = sid * ept + j * _EB
                pltpu.sync_copy(row_h.at[pl.ds(base, _EB)], ri)
                pltpu.sync_copy(crd_h.at[pl.ds(base, _EB)], cb16)

                def expand(i, c):
                    mb[i, pl.ds(0, _XW)] = cb16[i]
                    return c
                lax.fori_loop(0, _EB, expand, 0)
                pltpu.sync_copy(mb, acc.at[ri], add=True)
                return carry
            lax.fori_loop(0, nch, body, 0)

        plsc.subcore_barrier()

        def w1(t, c):
            pltpu.sync_copy(acc.at[pl.ds(rbase + t * 8, 8)], stg)

            @pl.when(cid == 0)
            def _():
                pltpu.sync_copy(stg, agg_o.at[pl.ds(rbase + t * 8, 8)])

            @pl.when(cid == 1)
            def _():
                pltpu.sync_copy(stg, cagg_o.at[pl.ds(rbase + t * 8, 8)])
            return c
        lax.fori_loop(0, npt // 8, w1, 0)

    return k(row, msg, coordc)


def _node_update(h, agg_a, agg_b, cagg_a, cagg_b, xp, wn1h, wn1a, b_n1, w_n2,
                 b_n2, gamma, beta, inv_scale):
    n, d = h.shape
    hdim = wn1h.shape[1]
    bn = 1000

    def body(h_ref, aa_ref, ab_ref, ca_ref, cb_ref, xp_ref, w1h_ref, w1a_ref,
             b1_ref, w2_ref, b2_ref, g_ref, bt_ref, hout_ref, xout_ref):
        hb = h_ref[...]
        agg = aa_ref[...] + ab_ref[...]
        t = (jnp.dot(hb, w1h_ref[...], preferred_element_type=jnp.float32)
             + jnp.dot(agg, w1a_ref[...],
                       preferred_element_type=jnp.float32)
             + b1_ref[...])
        t = _silu(t)
        hn = jnp.dot(t, w2_ref[...], preferred_element_type=jnp.float32) + b2_ref[...]
        y = hb + hn
        mu = jnp.mean(y, axis=1, keepdims=True)
        yc = y - mu
        var = jnp.mean(yc * yc, axis=1, keepdims=True)
        hout_ref[...] = yc * jax.lax.rsqrt(var + 1e-05) * g_ref[...] + bt_ref[...]
        xout_ref[...] = (xp_ref[...]
                         + (ca_ref[:, :_XW] + cb_ref[:, :_XW]) * inv_scale)

    full = lambda s: pl.BlockSpec(s, lambda i: (0, 0))
    return pl.pallas_call(
        body,
        grid=(n // bn,),
        in_specs=[
            pl.BlockSpec((bn, d), lambda i: (i, 0)),
            pl.BlockSpec((bn, hdim), lambda i: (i, 0)),
            pl.BlockSpec((bn, hdim), lambda i: (i, 0)),
            pl.BlockSpec((bn, hdim), lambda i: (i, 0)),
            pl.BlockSpec((bn, hdim), lambda i: (i, 0)),
            pl.BlockSpec((bn, _XW), lambda i: (i, 0)),
            full((d, hdim)), full((hdim, hdim)), full((1, hdim)),
            full((hdim, d)), full((1, d)), full((1, d)), full((1, d)),
        ],
        out_specs=[
            pl.BlockSpec((bn, d), lambda i: (i, 0)),
            pl.BlockSpec((bn, _XW), lambda i: (i, 0)),
        ],
        out_shape=[
            jax.ShapeDtypeStruct((n, d), jnp.float32),
            jax.ShapeDtypeStruct((n, _XW), jnp.float32),
        ],
    )(h, agg_a, agg_b, cagg_a, cagg_b, xp, wn1h, wn1a, b_n1, w_n2, b_n2,
      gamma, beta)


def kernel(h, x, edge_idx, edge_attr, W_e1, b_e1, W_e2, b_e2, W_att, b_att,
           W_n1, b_n1, W_n2, b_n2, W_c1, b_c1, W_c2, gamma, beta):
    n, d = h.shape
    e = edge_idx.shape[1]
    hdim = W_e2.shape[1]

    row = edge_idx[0].astype(jnp.int32)
    col = edge_idx[1].astype(jnp.int32)
    n_pad = ((n + 127) // 128) * 128
    xp = jnp.pad(x, ((0, n_pad - n), (0, 128 - x.shape[1])))
    xp16 = xp[:n, :_XW]

    w1a = W_e1[:d]
    w1b = W_e1[d:2 * d]
    w_d = W_e1[2 * d:2 * d + 1]
    w_ea = W_e1[2 * d + 1:]
    w_att_r = W_att.T
    w_c2_r = W_c2.T
    wn1h = W_n1[:d]
    wn1a = W_n1[d:]

    a_tab, b_tab = _precompute_ab(h, w1a, w1b)

    grp = _NW * _EB
    e0 = ((e // grp) // 2) * grp
    mlp_args = (edge_attr, w_ea, w_d, b_e1.reshape(1, -1),
                W_e2, b_e2.reshape(1, -1), w_att_r, b_att.reshape(1, 1), W_c1,
                b_c1.reshape(1, -1), w_c2_r)

    s0, d0 = _sc_gather(row[:e0], col[:e0], a_tab, b_tab, xp)
    s1, d1 = _sc_gather(row[e0:], col[e0:], a_tab, b_tab, xp)
    msg0, crd0 = _edge_mlp(s0, d0, edge_attr[:e0], *mlp_args[1:])
    msg1, crd1 = _edge_mlp(s1, d1, edge_attr[e0:], *mlp_args[1:])
    agg_a, cagg_a = _sc_scatter(row[:e0], msg0, crd0, n_pad)
    agg_b, cagg_b = _sc_scatter(row[e0:], msg1, crd1, n_pad)

    inv_scale = 1.0 / (e / n + _EPS)
    h_out, x_out_p = _node_update(
        h, agg_a[:n], agg_b[:n], cagg_a[:n], cagg_b[:n], xp16, wn1h, wn1a,
        b_n1.reshape(1, -1), W_n2, b_n2.reshape(1, -1), gamma.reshape(1, -1),
        beta.reshape(1, -1), inv_scale)
    return (h_out, x_out_p[:, :x.shape[1]])


# speedup vs baseline: 3.9258x; 1.0857x over previous
import functools

import jax
import jax.numpy as jnp
from jax import lax
from jax.experimental import pallas as pl
from jax.experimental.pallas import tpu as pltpu
from jax.experimental.pallas import tpu_sc as plsc

_EPS = 1e-08
_XW = 16
_EB = 80
_NW = 32
_NC = 2


def _silu(v):
    return v * jax.nn.sigmoid(v)


def _precompute_ab(h, w1a, w1b):
    n, d = h.shape
    bn = 1000
    def body(h_ref, wa_ref, wb_ref, a_ref, b_ref):
        hb = h_ref[...]
        a_ref[...] = jnp.dot(hb, wa_ref[...], preferred_element_type=jnp.float32)
        b_ref[...] = jnp.dot(hb, wb_ref[...], preferred_element_type=jnp.float32)
    return pl.pallas_call(
        body,
        grid=(n // bn,),
        in_specs=[
            pl.BlockSpec((bn, d), lambda i: (i, 0)),
            pl.BlockSpec(w1a.shape, lambda i: (0, 0)),
            pl.BlockSpec(w1b.shape, lambda i: (0, 0)),
        ],
        out_specs=[pl.BlockSpec((bn, w1a.shape[1]), lambda i: (i, 0))] * 2,
        out_shape=[jax.ShapeDtypeStruct((n, w1a.shape[1]), jnp.float32)] * 2,
    )(h, w1a, w1b)


def _sc_gather(row, col, a_tab, b_tab, x_tab):
    e = row.shape[0]
    n, hdim = a_tab.shape
    xw = x_tab.shape[1]
    ept = e // _NW
    nch = ept // _EB
    mesh = plsc.VectorSubcoreMesh(core_axis_name="c", subcore_axis_name="s")

    @functools.partial(
        pl.kernel,
        mesh=mesh,
        out_type=(
            jax.ShapeDtypeStruct((e, hdim), jnp.float32),
            jax.ShapeDtypeStruct((e, _XW), jnp.float32),
        ),
        scratch_types=[
            pltpu.VMEM((_EB,), jnp.int32),
            pltpu.VMEM((_EB,), jnp.int32),
            pltpu.VMEM((_EB, hdim), jnp.float32),
            pltpu.VMEM((_EB, hdim), jnp.float32),
            pltpu.VMEM((_EB, xw), jnp.float32),
            pltpu.VMEM((_EB, xw), jnp.float32),
            pltpu.VMEM((_EB, _XW), jnp.float32),
            pltpu.VMEM((_EB,), jnp.int32),
            pltpu.VMEM((_EB,), jnp.int32),
            pltpu.VMEM((_EB, hdim), jnp.float32),
            pltpu.VMEM((_EB, hdim), jnp.float32),
            pltpu.VMEM((_EB, xw), jnp.float32),
            pltpu.VMEM((_EB, xw), jnp.float32),
            pltpu.VMEM((_EB, _XW), jnp.float32),
            pltpu.SemaphoreType.DMA,
            pltpu.SemaphoreType.DMA,
            pltpu.SemaphoreType.DMA,
        ],
    )
    def k(row_h, col_h, a_h, b_h, x_h, s_o, d_o,
          ri0, ci0, ab0, bb0, xr0, xc0, df0,
          ri1, ci1, ab1, bb1, xr1, xc1, df1,
          semA, semB, semW):
        wid = lax.axis_index("s") * _NC + lax.axis_index("c")
        base0 = wid * ept

        def addrows(ab, bb, xr, xc, df):
            def srow(i, c):
                for kk in range(hdim // 16):
                    sl = pl.ds(kk * 16, 16)
                    ab[i, sl] = ab[i, sl] + bb[i, sl]
                df[i] = xr[i, pl.ds(0, _XW)] - xc[i, pl.ds(0, _XW)]
                return c
            lax.fori_loop(0, _EB, srow, 0)

        def fire(b, ri, ci, ab, bb, xr, xc, sem):
            pltpu.sync_copy(row_h.at[pl.ds(b, _EB)], ri)
            pltpu.sync_copy(col_h.at[pl.ds(b, _EB)], ci)
            return (pltpu.async_copy(a_h.at[ri], ab, sem),
                    pltpu.async_copy(b_h.at[ci], bb, sem),
                    pltpu.async_copy(x_h.at[ri], xr, sem),
                    pltpu.async_copy(x_h.at[ci], xc, sem))

        def body(jj, carry):
            b0 = base0 + (2 * jj) * _EB
            b1 = b0 + _EB
            g0 = fire(b0, ri0, ci0, ab0, bb0, xr0, xc0, semA)
            g1 = fire(b1, ri1, ci1, ab1, bb1, xr1, xc1, semB)
            for c in g0:
                c.wait()
            addrows(ab0, bb0, xr0, xc0, df0)
            w0 = (pltpu.async_copy(ab0, s_o.at[pl.ds(b0, _EB)], semW),
                  pltpu.async_copy(df0, d_o.at[pl.ds(b0, _EB)], semW))
            for c in g1:
                c.wait()
            addrows(ab1, bb1, xr1, xc1, df1)
            w1 = (pltpu.async_copy(ab1, s_o.at[pl.ds(b1, _EB)], semW),
                  pltpu.async_copy(df1, d_o.at[pl.ds(b1, _EB)], semW))
            for c in w0 + w1:
                c.wait()
            return carry

        lax.fori_loop(0, nch // 2, body, 0)

        if nch % 2:
            bt = base0 + (nch - 1) * _EB
            gt = fire(bt, ri0, ci0, ab0, bb0, xr0, xc0, semA)
            for c in gt:
                c.wait()
            addrows(ab0, bb0, xr0, xc0, df0)
            pltpu.sync_copy(ab0, s_o.at[pl.ds(bt, _EB)])
            pltpu.sync_copy(df0, d_o.at[pl.ds(bt, _EB)])

    return k(row, col, a_tab, b_tab, x_tab)


def _edge_mlp(svec, dvec, eattr, w_ea, w_d, b_e1, w_e2, b_e2,
              w_att_r, b_att, w_c1, b_c1, w_c2_r):
    e, hdim = svec.shape
    ed = eattr.shape[1]
    be = 512

    def body(s_ref, d_ref, ea_ref, wea_ref, wd_ref, be1_ref,
             we2_ref, be2_ref, watt_ref, batt_ref, wc1_ref, bc1_ref, wc2_ref,
             msg_ref, crd_ref):
        d = d_ref[...]
        dist_sq = jnp.sum(d * d, axis=1, keepdims=True)
        pre1 = (s_ref[...] + dist_sq * wd_ref[...]
                + jnp.dot(ea_ref[...], wea_ref[...],
                          preferred_element_type=jnp.float32) + be1_ref[...])
        t = _silu(pre1)
        msg0 = _silu(jnp.dot(t, we2_ref[...],
                             preferred_element_type=jnp.float32) + be2_ref[...])
        att = jax.nn.sigmoid(
            jnp.sum(msg0 * watt_ref[...], axis=1, keepdims=True) + batt_ref[...])
        msg = msg0 * att
        u = _silu(jnp.dot(msg, wc1_ref[...],
                          preferred_element_type=jnp.float32) + bc1_ref[...])
        cw = jnp.tanh(jnp.sum(u * wc2_ref[...], axis=1, keepdims=True))
        unit = d * jax.lax.rsqrt(dist_sq + _EPS)
        msg_ref[...] = msg
        crd_ref[...] = cw * unit

    full = lambda s: pl.BlockSpec(s, lambda i: (0, 0))
    return pl.pallas_call(
        body,
        grid=(e // be,),
        in_specs=[
            pl.BlockSpec((be, hdim), lambda i: (i, 0)),
            pl.BlockSpec((be, _XW), lambda i: (i, 0)),
            pl.BlockSpec((be, ed), lambda i: (i, 0)),
            full((ed, hdim)), full((1, hdim)), full((1, hdim)),
            full((hdim, hdim)), full((1, hdim)),
            full((1, hdim)), full((1, 1)),
            full((hdim, hdim)), full((1, hdim)), full((1, hdim)),
        ],
        out_specs=[
            pl.BlockSpec((be, hdim), lambda i: (i, 0)),
            pl.BlockSpec((be, _XW), lambda i: (i, 0)),
        ],
        out_shape=[
            jax.ShapeDtypeStruct((e, hdim), jnp.float32),
            jax.ShapeDtypeStruct((e, _XW), jnp.float32),
        ],
    )(svec, dvec, eattr, w_ea, w_d, b_e1, w_e2, b_e2,
      w_att_r, b_att, w_c1, b_c1, w_c2_r)


def _sc_scatter(row, msg, coordc, n_pad):
    e, hdim = msg.shape
    ns = _NW // _NC
    ept = e // ns
    nch = ept // _EB
    npt = n_pad // ns
    mesh = plsc.VectorSubcoreMesh(core_axis_name="c", subcore_axis_name="s")

    @functools.partial(
        pl.kernel,
        mesh=mesh,
        out_type=(
            jax.ShapeDtypeStruct((n_pad, hdim), jnp.float32),
            jax.ShapeDtypeStruct((n_pad, hdim), jnp.float32),
        ),
        scratch_types=[
            pltpu.VMEM((_EB,), jnp.int32),
            pltpu.VMEM((_EB, hdim), jnp.float32),
            pltpu.VMEM((_EB, _XW), jnp.float32),
            pltpu.VMEM((_EB,), jnp.int32),
            pltpu.VMEM((_EB, hdim), jnp.float32),
            pltpu.VMEM((_EB, _XW), jnp.float32),
            pltpu.VMEM((8, hdim), jnp.float32),
            pltpu.VMEM_SHARED((n_pad, hdim), jnp.float32),
            pltpu.SemaphoreType.DMA,
            pltpu.SemaphoreType.DMA,
            pltpu.SemaphoreType.DMA,
        ],
    )
    def k(row_h, msg_h, crd_h, agg_o, cagg_o,
          ri, mb, cb16, ri1, mb1, cb16b, stg, acc, semA, semB, semS):
        cid = lax.axis_index("c")
        sid = lax.axis_index("s")
        rbase = sid * npt

        for i in range(8):
            for kk in range(hdim // 16):
                stg[i, pl.ds(kk * 16, 16)] = jnp.zeros((16,), jnp.float32)

        def z3(t, c):
            pltpu.sync_copy(stg, acc.at[pl.ds(rbase + t * 8, 8)])
            return c
        lax.fori_loop(0, npt // 8, z3, 0)

        def z4(i, c):
            for kk in range(hdim // 16):
                mb[i, pl.ds(kk * 16, 16)] = jnp.zeros((16,), jnp.float32)
                mb1[i, pl.ds(kk * 16, 16)] = jnp.zeros((16,), jnp.float32)
            return c
        lax.fori_loop(0, _EB, z4, 0)
        plsc.subcore_barrier()

        @pl.when(cid == 0)
        def _():
            def body(jj, carry):
                b0 = sid * ept + (2 * jj) * _EB
                b1 = b0 + _EB
                l0 = (pltpu.async_copy(row_h.at[pl.ds(b0, _EB)], ri, semA),
                      pltpu.async_copy(msg_h.at[pl.ds(b0, _EB)], mb, semA))
                l1 = (pltpu.async_copy(row_h.at[pl.ds(b1, _EB)], ri1, semB),
                      pltpu.async_copy(msg_h.at[pl.ds(b1, _EB)], mb1, semB))
                for c in l0:
                    c.wait()
                s0 = pltpu.async_copy(mb, acc.at[ri], semS, add=True)
                for c in l1:
                    c.wait()
                s1 = pltpu.async_copy(mb1, acc.at[ri1], semS, add=True)
                s0.wait()
                s1.wait()
                return carry
            lax.fori_loop(0, nch // 2, body, 0)

        @pl.when(cid == 1)
        def _():
            def body(jj, carry):
                b0 = sid * ept + (2 * jj) * _EB
                b1 = b0 + _EB
                l0 = (pltpu.async_copy(row_h.at[pl.ds(b0, _EB)], ri, semA),
                      pltpu.async_copy(crd_h.at[pl.ds(b0, _EB)], cb16, semA))
                l1 = (pltpu.async_copy(row_h.at[pl.ds(b1, _EB)], ri1, semB),
                      pltpu.async_copy(crd_h.at[pl.ds(b1, _EB)], cb16b, semB))
                for c in l0:
                    c.wait()

                def expand(i, c):
                    mb[i, pl.ds(0, _XW)] = cb16[i]
                    return c
                lax.fori_loop(0, _EB, expand, 0)
                s0 = pltpu.async_copy(mb, acc.at[ri], semS, add=True)
                for c in l1:
                    c.wait()

                def expand1(i, c):
                    mb1[i, pl.ds(0, _XW)] = cb16b[i]
                    return c
                lax.fori_loop(0, _EB, expand1, 0)
                s1 = pltpu.async_copy(mb1, acc.at[ri1], semS, add=True)
                s0.wait()
                s1.wait()
                return carry
            lax.fori_loop(0, nch // 2, body, 0)

        plsc.subcore_barrier()

        def w1(t, c):
            pltpu.sync_copy(acc.at[pl.ds(rbase + t * 8, 8)], stg)

            @pl.when(cid == 0)
            def _():
                pltpu.sync_copy(stg, agg_o.at[pl.ds(rbase + t * 8, 8)])

            @pl.when(cid == 1)
            def _():
                pltpu.sync_copy(stg, cagg_o.at[pl.ds(rbase + t * 8, 8)])
            return c
        lax.fori_loop(0, npt // 8, w1, 0)

    return k(row, msg, coordc)


def _node_update(h, agg_a, agg_b, cagg_a, cagg_b, xp, wn1h, wn1a, b_n1, w_n2,
                 b_n2, gamma, beta, inv_scale):
    n, d = h.shape
    hdim = wn1h.shape[1]
    bn = 1000

    def body(h_ref, aa_ref, ab_ref, ca_ref, cb_ref, xp_ref, w1h_ref, w1a_ref,
             b1_ref, w2_ref, b2_ref, g_ref, bt_ref, hout_ref, xout_ref):
        hb = h_ref[...]
        agg = aa_ref[...] + ab_ref[...]
        t = (jnp.dot(hb, w1h_ref[...], preferred_element_type=jnp.float32)
             + jnp.dot(agg, w1a_ref[...],
                       preferred_element_type=jnp.float32)
             + b1_ref[...])
        t = _silu(t)
        hn = jnp.dot(t, w2_ref[...], preferred_element_type=jnp.float32) + b2_ref[...]
        y = hb + hn
        mu = jnp.mean(y, axis=1, keepdims=True)
        yc = y - mu
        var = jnp.mean(yc * yc, axis=1, keepdims=True)
        hout_ref[...] = yc * jax.lax.rsqrt(var + 1e-05) * g_ref[...] + bt_ref[...]
        xout_ref[...] = (xp_ref[...]
                         + (ca_ref[:, :_XW] + cb_ref[:, :_XW]) * inv_scale)

    full = lambda s: pl.BlockSpec(s, lambda i: (0, 0))
    return pl.pallas_call(
        body,
        grid=(n // bn,),
        in_specs=[
            pl.BlockSpec((bn, d), lambda i: (i, 0)),
            pl.BlockSpec((bn, hdim), lambda i: (i, 0)),
            pl.BlockSpec((bn, hdim), lambda i: (i, 0)),
            pl.BlockSpec((bn, hdim), lambda i: (i, 0)),
            pl.BlockSpec((bn, hdim), lambda i: (i, 0)),
            pl.BlockSpec((bn, _XW), lambda i: (i, 0)),
            full((d, hdim)), full((hdim, hdim)), full((1, hdim)),
            full((hdim, d)), full((1, d)), full((1, d)), full((1, d)),
        ],
        out_specs=[
            pl.BlockSpec((bn, d), lambda i: (i, 0)),
            pl.BlockSpec((bn, _XW), lambda i: (i, 0)),
        ],
        out_shape=[
            jax.ShapeDtypeStruct((n, d), jnp.float32),
            jax.ShapeDtypeStruct((n, _XW), jnp.float32),
        ],
    )(h, agg_a, agg_b, cagg_a, cagg_b, xp, wn1h, wn1a, b_n1, w_n2, b_n2,
      gamma, beta)


def kernel(h, x, edge_idx, edge_attr, W_e1, b_e1, W_e2, b_e2, W_att, b_att,
           W_n1, b_n1, W_n2, b_n2, W_c1, b_c1, W_c2, gamma, beta):
    n, d = h.shape
    e = edge_idx.shape[1]
    hdim = W_e2.shape[1]

    row = edge_idx[0].astype(jnp.int32)
    col = edge_idx[1].astype(jnp.int32)
    n_pad = ((n + 127) // 128) * 128
    xp = jnp.pad(x, ((0, n_pad - n), (0, 128 - x.shape[1])))
    xp16 = xp[:n, :_XW]

    w1a = W_e1[:d]
    w1b = W_e1[d:2 * d]
    w_d = W_e1[2 * d:2 * d + 1]
    w_ea = W_e1[2 * d + 1:]
    w_att_r = W_att.T
    w_c2_r = W_c2.T
    wn1h = W_n1[:d]
    wn1a = W_n1[d:]

    a_tab, b_tab = _precompute_ab(h, w1a, w1b)

    grp = _NW * _EB
    e0 = ((e // grp) // 2) * grp
    mlp_args = (edge_attr, w_ea, w_d, b_e1.reshape(1, -1),
                W_e2, b_e2.reshape(1, -1), w_att_r, b_att.reshape(1, 1), W_c1,
                b_c1.reshape(1, -1), w_c2_r)

    s0, d0 = _sc_gather(row[:e0], col[:e0], a_tab, b_tab, xp)
    s1, d1 = _sc_gather(row[e0:], col[e0:], a_tab, b_tab, xp)
    msg0, crd0 = _edge_mlp(s0, d0, edge_attr[:e0], *mlp_args[1:])
    msg1, crd1 = _edge_mlp(s1, d1, edge_attr[e0:], *mlp_args[1:])
    agg_a, cagg_a = _sc_scatter(row[:e0], msg0, crd0, n_pad)
    agg_b, cagg_b = _sc_scatter(row[e0:], msg1, crd1, n_pad)

    inv_scale = 1.0 / (e / n + _EPS)
    h_out, x_out_p = _node_update(
        h, agg_a[:n], agg_b[:n], cagg_a[:n], cagg_b[:n], xp16, wn1h, wn1a,
        b_n1.reshape(1, -1), W_n2, b_n2.reshape(1, -1), gamma.reshape(1, -1),
        beta.reshape(1, -1), inv_scale)
    return (h_out, x_out_p[:, :x.shape[1]])


# speedup vs baseline: 4.2472x; 1.0819x over previous
import functools

import jax
import jax.numpy as jnp
from jax import lax
from jax.experimental import pallas as pl
from jax.experimental.pallas import tpu as pltpu
from jax.experimental.pallas import tpu_sc as plsc

_EPS = 1e-08
_XW = 16
_EB = 80
_NW = 32
_NC = 2


def _silu(v):
    return v * jax.nn.sigmoid(v)


def _precompute_ab(h, w1a, w1b):
    n, d = h.shape
    bn = 1000
    def body(h_ref, wa_ref, wb_ref, a_ref, b_ref):
        hb = h_ref[...]
        a_ref[...] = jnp.dot(hb, wa_ref[...], preferred_element_type=jnp.float32)
        b_ref[...] = jnp.dot(hb, wb_ref[...], preferred_element_type=jnp.float32)
    return pl.pallas_call(
        body,
        grid=(n // bn,),
        in_specs=[
            pl.BlockSpec((bn, d), lambda i: (i, 0)),
            pl.BlockSpec(w1a.shape, lambda i: (0, 0)),
            pl.BlockSpec(w1b.shape, lambda i: (0, 0)),
        ],
        out_specs=[pl.BlockSpec((bn, w1a.shape[1]), lambda i: (i, 0))] * 2,
        out_shape=[jax.ShapeDtypeStruct((n, w1a.shape[1]), jnp.float32)] * 2,
    )(h, w1a, w1b)


def _sc_gather(row, col, a_tab, b_tab, x_tab):
    e = row.shape[0]
    n, hdim = a_tab.shape
    xw = x_tab.shape[1]
    ept = e // _NW
    nch = ept // _EB
    mesh = plsc.VectorSubcoreMesh(core_axis_name="c", subcore_axis_name="s")

    @functools.partial(
        pl.kernel,
        mesh=mesh,
        out_type=(
            jax.ShapeDtypeStruct((e, hdim), jnp.float32),
            jax.ShapeDtypeStruct((e, _XW), jnp.float32),
        ),
        scratch_types=[
            pltpu.VMEM((_EB,), jnp.int32),
            pltpu.VMEM((_EB,), jnp.int32),
            pltpu.VMEM((_EB, hdim), jnp.float32),
            pltpu.VMEM((_EB, hdim), jnp.float32),
            pltpu.VMEM((_EB, xw), jnp.float32),
            pltpu.VMEM((_EB, xw), jnp.float32),
            pltpu.VMEM((_EB, _XW), jnp.float32),
            pltpu.VMEM((_EB,), jnp.int32),
            pltpu.VMEM((_EB,), jnp.int32),
            pltpu.VMEM((_EB, hdim), jnp.float32),
            pltpu.VMEM((_EB, hdim), jnp.float32),
            pltpu.VMEM((_EB, xw), jnp.float32),
            pltpu.VMEM((_EB, xw), jnp.float32),
            pltpu.VMEM((_EB, _XW), jnp.float32),
            pltpu.SemaphoreType.DMA,
            pltpu.SemaphoreType.DMA,
            pltpu.SemaphoreType.DMA,
        ],
    )
    def k(row_h, col_h, a_h, b_h, x_h, s_o, d_o,
          ri0, ci0, ab0, bb0, xr0, xc0, df0,
          ri1, ci1, ab1, bb1, xr1, xc1, df1,
          semA, semB, semW):
        wid = lax.axis_index("s") * _NC + lax.axis_index("c")
        base0 = wid * ept

        def addrows(ab, bb, xr, xc, df):
            def srow(i, c):
                for kk in range(hdim // 16):
                    sl = pl.ds(kk * 16, 16)
                    ab[i, sl] = ab[i, sl] + bb[i, sl]
                df[i] = xr[i, pl.ds(0, _XW)] - xc[i, pl.ds(0, _XW)]
                return c
            lax.fori_loop(0, _EB, srow, 0)

        def fire(b, ri, ci, ab, bb, xr, xc, sem):
            pltpu.sync_copy(row_h.at[pl.ds(b, _EB)], ri)
            pltpu.sync_copy(col_h.at[pl.ds(b, _EB)], ci)
            return (pltpu.async_copy(a_h.at[ri], ab, sem),
                    pltpu.async_copy(b_h.at[ci], bb, sem),
                    pltpu.async_copy(x_h.at[ri], xr, sem),
                    pltpu.async_copy(x_h.at[ci], xc, sem))

        def body(jj, carry):
            b0 = base0 + (2 * jj) * _EB
            b1 = b0 + _EB
            g0 = fire(b0, ri0, ci0, ab0, bb0, xr0, xc0, semA)
            g1 = fire(b1, ri1, ci1, ab1, bb1, xr1, xc1, semB)
            for c in g0:
                c.wait()
            addrows(ab0, bb0, xr0, xc0, df0)
            w0 = (pltpu.async_copy(ab0, s_o.at[pl.ds(b0, _EB)], semW),
                  pltpu.async_copy(df0, d_o.at[pl.ds(b0, _EB)], semW))
            for c in g1:
                c.wait()
            addrows(ab1, bb1, xr1, xc1, df1)
            w1 = (pltpu.async_copy(ab1, s_o.at[pl.ds(b1, _EB)], semW),
                  pltpu.async_copy(df1, d_o.at[pl.ds(b1, _EB)], semW))
            for c in w0 + w1:
                c.wait()
            return carry

        lax.fori_loop(0, nch // 2, body, 0)

        if nch % 2:
            bt = base0 + (nch - 1) * _EB
            gt = fire(bt, ri0, ci0, ab0, bb0, xr0, xc0, semA)
            for c in gt:
                c.wait()
            addrows(ab0, bb0, xr0, xc0, df0)
            pltpu.sync_copy(ab0, s_o.at[pl.ds(bt, _EB)])
            pltpu.sync_copy(df0, d_o.at[pl.ds(bt, _EB)])

    return k(row, col, a_tab, b_tab, x_tab)


def _edge_mlp(svec, dvec, eattr, w_ea, w_d, b_e1, w_e2, b_e2,
              w_att_r, b_att, w_c1, b_c1, w_c2_r):
    e, hdim = svec.shape
    ed = eattr.shape[1]
    be = 512

    def body(s_ref, d_ref, ea_ref, wea_ref, wd_ref, be1_ref,
             we2_ref, be2_ref, watt_ref, batt_ref, wc1_ref, bc1_ref, wc2_ref,
             msg_ref, crd_ref):
        d = d_ref[...]
        dist_sq = jnp.sum(d * d, axis=1, keepdims=True)
        pre1 = (s_ref[...] + dist_sq * wd_ref[...]
                + jnp.dot(ea_ref[...], wea_ref[...],
                          preferred_element_type=jnp.float32) + be1_ref[...])
        t = _silu(pre1)
        msg0 = _silu(jnp.dot(t, we2_ref[...],
                             preferred_element_type=jnp.float32) + be2_ref[...])
        att = jax.nn.sigmoid(
            jnp.sum(msg0 * watt_ref[...], axis=1, keepdims=True) + batt_ref[...])
        msg = msg0 * att
        u = _silu(jnp.dot(msg, wc1_ref[...],
                          preferred_element_type=jnp.float32) + bc1_ref[...])
        cw = jnp.tanh(jnp.sum(u * wc2_ref[...], axis=1, keepdims=True))
        unit = d * jax.lax.rsqrt(dist_sq + _EPS)
        msg_ref[...] = msg
        crd_ref[...] = cw * unit

    full = lambda s: pl.BlockSpec(s, lambda i: (0, 0))
    return pl.pallas_call(
        body,
        grid=(e // be,),
        in_specs=[
            pl.BlockSpec((be, hdim), lambda i: (i, 0)),
            pl.BlockSpec((be, _XW), lambda i: (i, 0)),
            pl.BlockSpec((be, ed), lambda i: (i, 0)),
            full((ed, hdim)), full((1, hdim)), full((1, hdim)),
            full((hdim, hdim)), full((1, hdim)),
            full((1, hdim)), full((1, 1)),
            full((hdim, hdim)), full((1, hdim)), full((1, hdim)),
        ],
        out_specs=[
            pl.BlockSpec((be, hdim), lambda i: (i, 0)),
            pl.BlockSpec((be, _XW), lambda i: (i, 0)),
        ],
        out_shape=[
            jax.ShapeDtypeStruct((e, hdim), jnp.float32),
            jax.ShapeDtypeStruct((e, _XW), jnp.float32),
        ],
    )(svec, dvec, eattr, w_ea, w_d, b_e1, w_e2, b_e2,
      w_att_r, b_att, w_c1, b_c1, w_c2_r)


def _sc_scatter(row, msg, coordc, n_pad):
    e, hdim = msg.shape
    ns = _NW // _NC
    ept = e // ns
    nch = ept // _EB
    npt = n_pad // ns
    mesh = plsc.VectorSubcoreMesh(core_axis_name="c", subcore_axis_name="s")

    @functools.partial(
        pl.kernel,
        mesh=mesh,
        out_type=(
            jax.ShapeDtypeStruct((n_pad, hdim), jnp.float32),
            jax.ShapeDtypeStruct((n_pad, hdim), jnp.float32),
        ),
        scratch_types=[
            pltpu.VMEM((_EB,), jnp.int32),
            pltpu.VMEM((_EB, hdim), jnp.float32),
            pltpu.VMEM((_EB, _XW), jnp.float32),
            pltpu.VMEM((_EB,), jnp.int32),
            pltpu.VMEM((_EB, hdim), jnp.float32),
            pltpu.VMEM((_EB, _XW), jnp.float32),
            pltpu.VMEM((8, hdim), jnp.float32),
            pltpu.VMEM_SHARED((n_pad, hdim), jnp.float32),
            pltpu.SemaphoreType.DMA,
            pltpu.SemaphoreType.DMA,
            pltpu.SemaphoreType.DMA,
        ],
    )
    def k(row_h, msg_h, crd_h, agg_o, cagg_o,
          ri, mb, cb16, ri1, mb1, cb16b, stg, acc, semA, semB, semS):
        cid = lax.axis_index("c")
        sid = lax.axis_index("s")
        rbase = sid * npt

        for i in range(8):
            for kk in range(hdim // 16):
                stg[i, pl.ds(kk * 16, 16)] = jnp.zeros((16,), jnp.float32)

        def z3(t, c):
            pltpu.sync_copy(stg, acc.at[pl.ds(rbase + t * 8, 8)])
            return c
        lax.fori_loop(0, npt // 8, z3, 0)

        def z4(i, c):
            for kk in range(hdim // 16):
                mb[i, pl.ds(kk * 16, 16)] = jnp.zeros((16,), jnp.float32)
                mb1[i, pl.ds(kk * 16, 16)] = jnp.zeros((16,), jnp.float32)
            return c
        lax.fori_loop(0, _EB, z4, 0)
        plsc.subcore_barrier()

        @pl.when(cid == 0)
        def _():
            def body(jj, carry):
                b0 = sid * ept + (2 * jj) * _EB
                b1 = b0 + _EB
                l0 = (pltpu.async_copy(row_h.at[pl.ds(b0, _EB)], ri, semA),
                      pltpu.async_copy(msg_h.at[pl.ds(b0, _EB)], mb, semA))
                l1 = (pltpu.async_copy(row_h.at[pl.ds(b1, _EB)], ri1, semB),
                      pltpu.async_copy(msg_h.at[pl.ds(b1, _EB)], mb1, semB))
                for c in l0:
                    c.wait()
                s0 = pltpu.async_copy(mb, acc.at[ri], semS, add=True)
                for c in l1:
                    c.wait()
                s1 = pltpu.async_copy(mb1, acc.at[ri1], semS, add=True)
                s0.wait()
                s1.wait()
                return carry
            lax.fori_loop(0, nch // 2, body, 0)

        @pl.when(cid == 1)
        def _():
            def body(jj, carry):
                b0 = sid * ept + (2 * jj) * _EB
                b1 = b0 + _EB
                l0 = (pltpu.async_copy(row_h.at[pl.ds(b0, _EB)], ri, semA),
                      pltpu.async_copy(crd_h.at[pl.ds(b0, _EB)], cb16, semA))
                l1 = (pltpu.async_copy(row_h.at[pl.ds(b1, _EB)], ri1, semB),
                      pltpu.async_copy(crd_h.at[pl.ds(b1, _EB)], cb16b, semB))
                for c in l0:
                    c.wait()

                def expand(i, c):
                    mb[i, pl.ds(0, _XW)] = cb16[i]
                    return c
                lax.fori_loop(0, _EB, expand, 0)
                s0 = pltpu.async_copy(mb, acc.at[ri], semS, add=True)
                for c in l1:
                    c.wait()

                def expand1(i, c):
                    mb1[i, pl.ds(0, _XW)] = cb16b[i]
                    return c
                lax.fori_loop(0, _EB, expand1, 0)
                s1 = pltpu.async_copy(mb1, acc.at[ri1], semS, add=True)
                s0.wait()
                s1.wait()
                return carry
            lax.fori_loop(0, nch // 2, body, 0)

        plsc.subcore_barrier()

        def w1(t, c):
            pltpu.sync_copy(acc.at[pl.ds(rbase + t * 8, 8)], stg)

            @pl.when(cid == 0)
            def _():
                pltpu.sync_copy(stg, agg_o.at[pl.ds(rbase + t * 8, 8)])

            @pl.when(cid == 1)
            def _():
                pltpu.sync_copy(stg, cagg_o.at[pl.ds(rbase + t * 8, 8)])
            return c
        lax.fori_loop(0, npt // 8, w1, 0)

    return k(row, msg, coordc)


def _node_update(h, agg_a, cagg_a, xp, wn1h, wn1a, b_n1, w_n2,
                 b_n2, gamma, beta, inv_scale):
    n, d = h.shape
    hdim = wn1h.shape[1]
    bn = 1000

    k = len(agg_a)

    def body(*refs):
        h_ref = refs[0]
        agg_refs = refs[1:1 + k]
        cagg_refs = refs[1 + k:1 + 2 * k]
        (xp_ref, w1h_ref, w1a_ref, b1_ref, w2_ref, b2_ref, g_ref,
         bt_ref, hout_ref, xout_ref) = refs[1 + 2 * k:]
        hb = h_ref[...]
        agg = agg_refs[0][...]
        for r in agg_refs[1:]:
            agg = agg + r[...]
        cagg = cagg_refs[0][:, :_XW]
        for r in cagg_refs[1:]:
            cagg = cagg + r[:, :_XW]
        t = (jnp.dot(hb, w1h_ref[...], preferred_element_type=jnp.float32)
             + jnp.dot(agg, w1a_ref[...],
                       preferred_element_type=jnp.float32)
             + b1_ref[...])
        t = _silu(t)
        hn = jnp.dot(t, w2_ref[...], preferred_element_type=jnp.float32) + b2_ref[...]
        y = hb + hn
        mu = jnp.mean(y, axis=1, keepdims=True)
        yc = y - mu
        var = jnp.mean(yc * yc, axis=1, keepdims=True)
        hout_ref[...] = yc * jax.lax.rsqrt(var + 1e-05) * g_ref[...] + bt_ref[...]
        xout_ref[...] = xp_ref[...] + cagg * inv_scale

    full = lambda s: pl.BlockSpec(s, lambda i: (0, 0))
    return pl.pallas_call(
        body,
        grid=(n // bn,),
        in_specs=[
            pl.BlockSpec((bn, d), lambda i: (i, 0)),
        ] + [pl.BlockSpec((bn, hdim), lambda i: (i, 0))] * (2 * k) + [
            pl.BlockSpec((bn, _XW), lambda i: (i, 0)),
            full((d, hdim)), full((hdim, hdim)), full((1, hdim)),
            full((hdim, d)), full((1, d)), full((1, d)), full((1, d)),
        ],
        out_specs=[
            pl.BlockSpec((bn, d), lambda i: (i, 0)),
            pl.BlockSpec((bn, _XW), lambda i: (i, 0)),
        ],
        out_shape=[
            jax.ShapeDtypeStruct((n, d), jnp.float32),
            jax.ShapeDtypeStruct((n, _XW), jnp.float32),
        ],
    )(h, *agg_a, *cagg_a, xp, wn1h, wn1a, b_n1, w_n2, b_n2,
      gamma, beta)


def kernel(h, x, edge_idx, edge_attr, W_e1, b_e1, W_e2, b_e2, W_att, b_att,
           W_n1, b_n1, W_n2, b_n2, W_c1, b_c1, W_c2, gamma, beta):
    n, d = h.shape
    e = edge_idx.shape[1]
    hdim = W_e2.shape[1]

    row = edge_idx[0].astype(jnp.int32)
    col = edge_idx[1].astype(jnp.int32)
    n_pad = ((n + 127) // 128) * 128
    xp = jnp.pad(x, ((0, n_pad - n), (0, 128 - x.shape[1])))
    xp16 = xp[:n, :_XW]

    w1a = W_e1[:d]
    w1b = W_e1[d:2 * d]
    w_d = W_e1[2 * d:2 * d + 1]
    w_ea = W_e1[2 * d + 1:]
    w_att_r = W_att.T
    w_c2_r = W_c2.T
    wn1h = W_n1[:d]
    wn1a = W_n1[d:]

    a_tab, b_tab = _precompute_ab(h, w1a, w1b)

    grp = _NW * _EB
    ng = e // grp
    small = max(1, ng // 8)
    mid = (ng - 2 * small) // 2
    parts_g = [small, mid, ng - 2 * small - mid, small]
    mlp_args = (w_ea, w_d, b_e1.reshape(1, -1),
                W_e2, b_e2.reshape(1, -1), w_att_r, b_att.reshape(1, 1), W_c1,
                b_c1.reshape(1, -1), w_c2_r)

    aggs, caggs = [], []
    off = 0
    for p in parts_g:
        sz = p * grp
        sl = slice(off, off + sz)
        sv, dv = _sc_gather(row[sl], col[sl], a_tab, b_tab, xp)
        msg_p, crd_p = _edge_mlp(sv, dv, edge_attr[sl], *mlp_args)
        agg_p, cagg_p = _sc_scatter(row[sl], msg_p, crd_p, n_pad)
        aggs.append(agg_p[:n])
        caggs.append(cagg_p[:n])
        off += sz

    inv_scale = 1.0 / (e / n + _EPS)
    h_out, x_out_p = _node_update(
        h, aggs, caggs, xp16, wn1h, wn1a,
        b_n1.reshape(1, -1), W_n2, b_n2.reshape(1, -1), gamma.reshape(1, -1),
        beta.reshape(1, -1), inv_scale)
    return (h_out, x_out_p[:, :x.shape[1]])
